# Initial kernel scaffold; baseline (speedup 1.0000x reference)
import jax
import jax.numpy as jnp
from jax.experimental import pallas as pl


def kernel(x):
    raise NotImplementedError("write your pallas kernel here")


# speedup vs baseline: 2.7713x; 2.7713x over previous
import jax
import jax.numpy as jnp
from jax import lax
from jax.experimental import pallas as pl
from jax.experimental.pallas import tpu as pltpu

K_P = 512
DIM = 6
N_PTS = 32768
K_NN = 16
ITERS = 7
BN = 2048
NBLK = N_PTS // BN

_BF = jnp.bfloat16


def _dotg(a, b, ca, cb):
    return lax.dot_general(
        a, b, (((ca,), (cb,)), ((), ())), preferred_element_type=jnp.float32
    )


def _dot_lp(a, b, ca, cb):
    return _dotg(a.astype(_BF), b.astype(_BF), ca, cb)


def _split3(a):
    a1 = a.astype(_BF).astype(jnp.float32)
    r = a - a1
    a2 = r.astype(_BF).astype(jnp.float32)
    a3 = r - a2
    return a1.astype(_BF), a2.astype(_BF), a3.astype(_BF)


def _dot_e_a01(a01, b, ca, cb):
    a = a01.astype(_BF)
    b1, b2, b3 = _split3(b)
    return _dotg(a, b1, ca, cb) + _dotg(a, b2, ca, cb) + _dotg(a, b3, ca, cb)


def _dot_e_b01(a, b01, ca, cb):
    b = b01.astype(_BF)
    a1, a2, a3 = _split3(a)
    return _dotg(a1, b, ca, cb) + _dotg(a2, b, ca, cb) + _dotg(a3, b, ca, cb)


def _kernel(pts_ref, cent0_ref, out_ref):
    col_iota = lax.broadcasted_iota(jnp.int32, (BN, K_P), 1)
    eye = (lax.broadcasted_iota(jnp.int32, (K_P, K_P), 0)
           == lax.broadcasted_iota(jnp.int32, (K_P, K_P), 1))
    eye_f = eye.astype(jnp.float32)
    ones_bn = jnp.ones((BN, 1), jnp.float32)

    def kmeans_step(_, cent):
        cent_t = _dot_e_b01(cent, eye_f, 0, 0)
        cnorm = jnp.sum(cent_t * cent_t, axis=0, keepdims=True)

        def blk(j, carry):
            sums, counts = carry
            p = pts_ref[pl.ds(j * BN, BN), :]
            pn = jnp.sum(p * p, axis=1, keepdims=True)
            dot = _dot_lp(p, cent, 1, 1)
            d = (pn - 2.0 * dot) + cnorm
            dmin = jnp.min(d, axis=1, keepdims=True)
            cand = jnp.where(d == dmin, col_iota, K_P)
            lab = jnp.min(cand, axis=1, keepdims=True)
            oh = (col_iota == lab).astype(jnp.float32)
            sums = sums + _dot_e_a01(oh, p, 0, 0)
            counts = counts + _dot_lp(oh, ones_bn, 0, 0)
            return sums, counts

        sums, counts = lax.fori_loop(
            0, NBLK, blk,
            (jnp.zeros((K_P, DIM), jnp.float32), jnp.zeros((K_P, 1), jnp.float32)),
        )
        newc = sums / jnp.maximum(counts, 1.0)
        return jnp.where(counts > 0, newc, cent)

    mc = lax.fori_loop(0, ITERS, kmeans_step, cent0_ref[...])

    mct = _dot_e_b01(mc, eye_f, 0, 0)
    n_col = jnp.sum(mc * mc, axis=1, keepdims=True)
    n_row = jnp.sum(mct * mct, axis=0, keepdims=True)
    d2 = (n_col - 2.0 * _dot_lp(mc, mc, 1, 1)) + n_row

    kcol_iota = lax.broadcasted_iota(jnp.int32, (K_P, K_P), 1)

    def pick(_, carry):
        ht, dd = carry
        dmin = jnp.min(dd, axis=1, keepdims=True)
        cand = jnp.where(dd == dmin, kcol_iota, K_P)
        sel = jnp.min(cand, axis=1, keepdims=True)
        onehot = kcol_iota == sel
        ht = jnp.where(onehot, 1.0, ht)
        dd = jnp.where(onehot, jnp.inf, dd)
        return ht, dd

    ht, _ = lax.fori_loop(
        0, K_NN, pick, (jnp.zeros((K_P, K_P), jnp.float32), d2)
    )

    dv_row = jnp.sum(ht, axis=0, keepdims=True)
    ht = jnp.where(eye & (dv_row == 0.0), 1.0, ht)

    deg_e = jnp.sum(ht, axis=1, keepdims=True)
    ones_col = jnp.ones((K_P, 1), jnp.float32)
    deg_v = _dot_lp(ht, ones_col, 0, 0)

    y_e = _dot_lp(ht, mc, 1, 0) / jnp.maximum(deg_e, 1.0)
    x_v = _dot_lp(ht, y_e, 0, 0) / jnp.maximum(deg_v, 1.0)
    out_ref[...] = x_v[:, :4]


@jax.jit
def kernel(x):
    pts = jnp.transpose(x, (0, 2, 1)).reshape(-1, DIM)
    cent0 = pts[:K_P]
    return pl.pallas_call(
        _kernel,
        out_shape=jax.ShapeDtypeStruct((K_P, 4), jnp.float32),
    )(pts, cent0)


# speedup vs baseline: 3.1582x; 1.1396x over previous
import jax
import jax.numpy as jnp
from jax import lax
from jax.experimental import pallas as pl
from jax.experimental.pallas import tpu as pltpu

K_P = 512
DIM = 6
N_PTS = 32768
K_NN = 16
ITERS = 7
BN = 2048
NBLK = N_PTS // BN

_BF = jnp.bfloat16


def _dotg(a, b, ca, cb):
    return lax.dot_general(
        a, b, (((ca,), (cb,)), ((), ())), preferred_element_type=jnp.float32
    )


def _dot_lp(a, b, ca, cb):
    return _dotg(a.astype(_BF), b.astype(_BF), ca, cb)


def _split3(a):
    a1 = a.astype(_BF).astype(jnp.float32)
    r = a - a1
    a2 = r.astype(_BF).astype(jnp.float32)
    a3 = r - a2
    return a1.astype(_BF), a2.astype(_BF), a3.astype(_BF)


def _dot_e_a01(a01, b, ca, cb):
    a = a01.astype(_BF)
    b1, b2, b3 = _split3(b)
    return _dotg(a, b1, ca, cb) + _dotg(a, b2, ca, cb) + _dotg(a, b3, ca, cb)


def _dot_e_b01(a, b01, ca, cb):
    b = b01.astype(_BF)
    a1, a2, a3 = _split3(a)
    return _dotg(a1, b, ca, cb) + _dotg(a2, b, ca, cb) + _dotg(a3, b, ca, cb)


def _kernel(pts_ref, cent0_ref, out_ref):
    col_iota = lax.broadcasted_iota(jnp.int32, (BN, K_P), 1).astype(jnp.float32)
    eye = (lax.broadcasted_iota(jnp.int32, (K_P, K_P), 0)
           == lax.broadcasted_iota(jnp.int32, (K_P, K_P), 1))
    eye_f = eye.astype(jnp.float32)
    ones_bn = jnp.ones((BN, 1), jnp.float32)

    def kmeans_step(_, cent):
        cent_t = _dot_e_b01(cent, eye_f, 0, 0)
        cnorm = jnp.sum(cent_t * cent_t, axis=0, keepdims=True)

        def blk(j, carry):
            sums, counts = carry
            p = pts_ref[pl.ds(j * BN, BN), :]
            pn = jnp.sum(p * p, axis=1, keepdims=True)
            dot = _dot_lp(p, cent, 1, 1)
            d = (pn - 2.0 * dot) + cnorm
            dmin = jnp.min(d, axis=1, keepdims=True)
            cand = jnp.where(d == dmin, col_iota, jnp.float32(K_P))
            lab = jnp.min(cand, axis=1, keepdims=True)
            oh = (col_iota == lab).astype(jnp.float32)
            sums = sums + _dot_e_a01(oh, p, 0, 0)
            counts = counts + _dot_lp(oh, ones_bn, 0, 0)
            return sums, counts

        sums, counts = lax.fori_loop(
            0, NBLK, blk,
            (jnp.zeros((K_P, DIM), jnp.float32), jnp.zeros((K_P, 1), jnp.float32)),
        )
        newc = sums / jnp.maximum(counts, 1.0)
        return jnp.where(counts > 0, newc, cent)

    mc = lax.fori_loop(0, ITERS, kmeans_step, cent0_ref[...])

    mct = _dot_e_b01(mc, eye_f, 0, 0)
    n_col = jnp.sum(mc * mc, axis=1, keepdims=True)
    n_row = jnp.sum(mct * mct, axis=0, keepdims=True)
    d2 = (n_col - 2.0 * _dot_lp(mc, mc, 1, 1)) + n_row

    kcol_iota = lax.broadcasted_iota(jnp.int32, (K_P, K_P), 1)

    def pick(_, carry):
        ht, dd = carry
        dmin = jnp.min(dd, axis=1, keepdims=True)
        cand = jnp.where(dd == dmin, kcol_iota, K_P)
        sel = jnp.min(cand, axis=1, keepdims=True)
        onehot = kcol_iota == sel
        ht = jnp.where(onehot, 1.0, ht)
        dd = jnp.where(onehot, jnp.inf, dd)
        return ht, dd

    ht, _ = lax.fori_loop(
        0, K_NN, pick, (jnp.zeros((K_P, K_P), jnp.float32), d2)
    )

    dv_row = jnp.sum(ht, axis=0, keepdims=True)
    ht = jnp.where(eye & (dv_row == 0.0), 1.0, ht)

    deg_e = jnp.sum(ht, axis=1, keepdims=True)
    ones_col = jnp.ones((K_P, 1), jnp.float32)
    deg_v = _dot_lp(ht, ones_col, 0, 0)

    y_e = _dot_lp(ht, mc, 1, 0) / jnp.maximum(deg_e, 1.0)
    x_v = _dot_lp(ht, y_e, 0, 0) / jnp.maximum(deg_v, 1.0)
    out_ref[...] = x_v[:, :4]


@jax.jit
def kernel(x):
    pts = jnp.transpose(x, (0, 2, 1)).reshape(-1, DIM)
    cent0 = pts[:K_P]
    return pl.pallas_call(
        _kernel,
        out_shape=jax.ShapeDtypeStruct((K_P, 4), jnp.float32),
    )(pts, cent0)


# speedup vs baseline: 3.1632x; 1.0016x over previous
import jax
import jax.numpy as jnp
from jax import lax
from jax.experimental import pallas as pl
from jax.experimental.pallas import tpu as pltpu

K_P = 512
DIM = 6
N_PTS = 32768
K_NN = 16
ITERS = 7
BN = 2048
NBLK = N_PTS // BN

_BF = jnp.bfloat16


def _dotg(a, b, ca, cb):
    return lax.dot_general(
        a, b, (((ca,), (cb,)), ((), ())), preferred_element_type=jnp.float32
    )


def _dot_lp(a, b, ca, cb):
    return _dotg(a.astype(_BF), b.astype(_BF), ca, cb)


def _split3(a):
    a1 = a.astype(_BF).astype(jnp.float32)
    r = a - a1
    a2 = r.astype(_BF).astype(jnp.float32)
    a3 = r - a2
    return a1.astype(_BF), a2.astype(_BF), a3.astype(_BF)


def _dot_e_a01(a01, b, ca, cb):
    a = a01.astype(_BF)
    b1, b2, b3 = _split3(b)
    return _dotg(a, b1, ca, cb) + _dotg(a, b2, ca, cb) + _dotg(a, b3, ca, cb)


def _dot_e_b01(a, b01, ca, cb):
    b = b01.astype(_BF)
    a1, a2, a3 = _split3(a)
    return _dotg(a1, b, ca, cb) + _dotg(a2, b, ca, cb) + _dotg(a3, b, ca, cb)


def _kernel(pts_ref, cent0_ref, out_ref):
    col_iota = lax.broadcasted_iota(jnp.int32, (BN, K_P), 1).astype(jnp.float32)
    eye = (lax.broadcasted_iota(jnp.int32, (K_P, K_P), 0)
           == lax.broadcasted_iota(jnp.int32, (K_P, K_P), 1))
    eye_f = eye.astype(jnp.float32)
    ones_bf = jnp.ones((BN, 1), _BF)

    def kmeans_step(_, cent):
        cent_t = _dot_e_b01(cent, eye_f, 0, 0)
        cnorm = jnp.sum(cent_t * cent_t, axis=0, keepdims=True)

        def blk(j, carry):
            sums, counts = carry
            p = pts_ref[pl.ds(j * BN, BN), :]
            pn = jnp.sum(p * p, axis=1, keepdims=True)
            dot = _dot_lp(p, cent, 1, 1)
            d = (pn - 2.0 * dot) + cnorm
            dmin = jnp.min(d, axis=1, keepdims=True)
            cand = jnp.where(d == dmin, col_iota, jnp.float32(K_P))
            lab = jnp.min(cand, axis=1, keepdims=True)
            oh = (col_iota == lab).astype(_BF)
            b1, b2, b3 = _split3(p)
            sums = sums + ((_dotg(oh, b1, 0, 0) + _dotg(oh, b2, 0, 0))
                           + _dotg(oh, b3, 0, 0))
            counts = counts + _dotg(oh, ones_bf, 0, 0)
            return sums, counts

        sums, counts = lax.fori_loop(
            0, NBLK, blk,
            (jnp.zeros((K_P, DIM), jnp.float32), jnp.zeros((K_P, 1), jnp.float32)),
        )
        newc = sums / jnp.maximum(counts, 1.0)
        return jnp.where(counts > 0, newc, cent)

    mc = lax.fori_loop(0, ITERS, kmeans_step, cent0_ref[...])

    mct = _dot_e_b01(mc, eye_f, 0, 0)
    n_col = jnp.sum(mc * mc, axis=1, keepdims=True)
    n_row = jnp.sum(mct * mct, axis=0, keepdims=True)
    d2 = (n_col - 2.0 * _dot_lp(mc, mc, 1, 1)) + n_row

    kcol_iota = lax.broadcasted_iota(jnp.int32, (K_P, K_P), 1)

    def pick(_, carry):
        ht, dd = carry
        dmin = jnp.min(dd, axis=1, keepdims=True)
        cand = jnp.where(dd == dmin, kcol_iota, K_P)
        sel = jnp.min(cand, axis=1, keepdims=True)
        onehot = kcol_iota == sel
        ht = jnp.where(onehot, 1.0, ht)
        dd = jnp.where(onehot, jnp.inf, dd)
        return ht, dd

    ht, _ = lax.fori_loop(
        0, K_NN, pick, (jnp.zeros((K_P, K_P), jnp.float32), d2)
    )

    dv_row = jnp.sum(ht, axis=0, keepdims=True)
    ht = jnp.where(eye & (dv_row == 0.0), 1.0, ht)

    deg_e = jnp.sum(ht, axis=1, keepdims=True)
    ones_col = jnp.ones((K_P, 1), jnp.float32)
    deg_v = _dot_lp(ht, ones_col, 0, 0)

    y_e = _dot_lp(ht, mc, 1, 0) / jnp.maximum(deg_e, 1.0)
    x_v = _dot_lp(ht, y_e, 0, 0) / jnp.maximum(deg_v, 1.0)
    out_ref[...] = x_v[:, :4]


@jax.jit
def kernel(x):
    pts = jnp.transpose(x, (0, 2, 1)).reshape(-1, DIM)
    cent0 = pts[:K_P]
    return pl.pallas_call(
        _kernel,
        out_shape=jax.ShapeDtypeStruct((K_P, 4), jnp.float32),
    )(pts, cent0)
